# baseline (speedup 1.0000x reference)
import functools

import jax
import jax.numpy as jnp
import numpy as np
from jax import lax
from jax.experimental import pallas as pl
from jax.experimental.pallas import tpu as pltpu
from jax.experimental.pallas import tpu_sc as plsc

_FIELDS = 26
_EMBED = 16
_NPAIRS = (_FIELDS * (_FIELDS - 1)) // 2
_IU_R, _IU_C = np.triu_indices(_FIELDS, k=1)

_NC = 2
_NS = 16
_NW = _NC * _NS
_G = 4
_BC = 16
_GPC = _BC // _G
_PD = _NPAIRS * _EMBED
_ROWW = _FIELDS * _EMBED


def _pairs_for_batch(rows_ref, ostage_ref, gb, lb_vec, iota16):
    rbase = gb * _FIELDS

    def body(p, carry):
        i, j = carry
        a = rows_ref[rbase + i, j, :]
        b = rows_ref[rbase + j, i, :]
        pd_vec = iota16 + p * _EMBED
        plsc.store_scatter(ostage_ref, [pd_vec, lb_vec], a * b)
        last = j == (_FIELDS - 1)
        i2 = jnp.where(last, i + 1, i)
        j2 = jnp.where(last, i + 2, j + 1)
        return (i2, j2)

    lax.fori_loop(0, _NPAIRS, body, (jnp.int32(0), jnp.int32(1)))


def _sc_body(nb, nchunk, x_hbm, v_hbm, out_hbm,
             idx_v, rows_v, ostage, gsem, osem):
    wid = lax.axis_index("s") * _NC + lax.axis_index("c")
    base = wid * nb
    iota16 = lax.iota(jnp.int32, _EMBED)

    def out_copy(c):
        return pltpu.make_async_copy(
            ostage, out_hbm.at[:, pl.ds((base + c * _BC), _BC)], osem)

    def chunk_body(c, carry):
        for lg in range(_GPC):
            g = c * _GPC + lg
            pltpu.sync_copy(
                x_hbm.at[pl.ds((base + g * _G) * _FIELDS, _G * _FIELDS)],
                idx_v)
            gather = pltpu.make_async_copy(v_hbm.at[idx_v], rows_v, gsem)
            gather.start()
            if lg == 0:
                @pl.when(c > 0)
                def _():
                    out_copy(c - 1).wait()
            gather.wait()

            def inner(gb, cc):
                lb_vec = jnp.broadcast_to(lg * _G + gb, (_EMBED,))
                _pairs_for_batch(rows_v, ostage, gb, lb_vec, iota16)
                return cc

            lax.fori_loop(0, _G, inner, 0)
        out_copy(c).start()
        return carry

    lax.fori_loop(0, nchunk, chunk_body, 0)
    out_copy(nchunk - 1).wait()


_VPW = 3128
_WINS = (0, 1568)
_WLEN = (1568, 1560)


def _t_body(vmax, vt_hbm, t2_hbm, stg0, stg1, trans0, trans1,
            isem0, isem1, osem0, osem1):
    stg = (stg0, stg1)
    trans = (trans0, trans1)
    isem = (isem0, isem1)
    osem = (osem0, osem1)
    wid = lax.axis_index("s") * _NC + lax.axis_index("c")
    iota16 = lax.iota(jnp.int32, _EMBED)
    r0w = jnp.minimum(wid * _VPW, vmax - _VPW)

    def in_copies(f, w):
        wl = _WLEN[w]
        base = r0w + _WINS[w]
        return [pltpu.make_async_copy(
                    vt_hbm.at[pl.ds((f * _EMBED + d) * vmax + base, wl)],
                    stg[w].at[pl.ds(d * wl, wl)],
                    isem[w])
                for d in range(_EMBED)]

    def out_copy(f, w):
        wl = _WLEN[w]
        return pltpu.make_async_copy(
            trans[w].at[pl.ds(0, wl), :],
            t2_hbm.at[pl.ds(r0w + _WINS[w], wl),
                      pl.ds(f * _EMBED, _EMBED)],
            osem[w])

    def start_in(f, w):
        for c in in_copies(f, w):
            c.start()

    def wait_in(f, w):
        for c in in_copies(f, w):
            c.wait()

    start_in(0, 0)

    def per_field(f, carry):
        for w in (0, 1):
            if w == 0:
                start_in(f, 1)
            else:
                @pl.when(f + 1 < _FIELDS)
                def _():
                    start_in(f + 1, 0)
            wait_in(f, w)

            @pl.when(f > 0)
            def _():
                out_copy(f - 1, w).wait()

            wl = _WLEN[w]

            def tr(r, cc, wl=wl, w=w):
                vals = plsc.load_gather(stg[w], [iota16 * wl + r])
                trans[w][r, :] = vals
                return cc

            lax.fori_loop(0, wl, tr, 0, unroll=4)
            out_copy(f, w).start()
        return carry

    lax.fori_loop(0, _FIELDS, per_field, 0)
    out_copy(_FIELDS - 1, 0).wait()
    out_copy(_FIELDS - 1, 1).wait()


def kernel(X, v):
    B, F = X.shape
    Vn, F2, D = v.shape
    assert F == _FIELDS and F2 == _FIELDS and D == _EMBED
    assert B % (_NW * _BC) == 0
    nb = B // _NW
    nchunk = nb // _BC

    x_flat = X.reshape(B * F).astype(jnp.int32)
    vt = jnp.transpose(v, (1, 2, 0)).reshape(-1)

    mesh = plsc.VectorSubcoreMesh(core_axis_name="c", subcore_axis_name="s")
    f32 = jnp.float32
    run_t = pl.kernel(
        functools.partial(_t_body, Vn),
        mesh=mesh,
        compiler_params=pltpu.CompilerParams(
            use_tc_tiling_on_sc=False, needs_layout_passes=False),
        out_type=jax.ShapeDtypeStruct((Vn, F * D), f32),
        scratch_types=[
            pltpu.VMEM((_EMBED * _WLEN[0],), f32),
            pltpu.VMEM((_EMBED * _WLEN[0],), f32),
            pltpu.VMEM((_WLEN[0], _EMBED), f32),
            pltpu.VMEM((_WLEN[0], _EMBED), f32),
            pltpu.SemaphoreType.DMA,
            pltpu.SemaphoreType.DMA,
            pltpu.SemaphoreType.DMA,
            pltpu.SemaphoreType.DMA,
        ],
    )
    run = pl.kernel(
        functools.partial(_sc_body, nb, nchunk),
        mesh=mesh,
        compiler_params=pltpu.CompilerParams(
            use_tc_tiling_on_sc=False, needs_layout_passes=False),
        out_type=jax.ShapeDtypeStruct((_PD, B), f32),
        scratch_types=[
            pltpu.VMEM((_G * _FIELDS,), jnp.int32),
            pltpu.VMEM((_G * _FIELDS, _FIELDS, _EMBED), f32),
            pltpu.VMEM((_PD, _BC), f32),
            pltpu.SemaphoreType.DMA,
            pltpu.SemaphoreType.DMA,
        ],
    )
    table2 = run_t(vt).reshape(Vn, F, D)
    out2 = run(x_flat, table2)
    return out2.reshape(_NPAIRS, _EMBED, B).transpose(2, 0, 1)


# speedup vs baseline: 2.3876x; 2.3876x over previous
import functools

import jax
import jax.numpy as jnp
import numpy as np
from jax import lax
from jax.experimental import pallas as pl
from jax.experimental.pallas import tpu as pltpu
from jax.experimental.pallas import tpu_sc as plsc

_FIELDS = 26
_EMBED = 16
_NPAIRS = (_FIELDS * (_FIELDS - 1)) // 2

_NC = 2
_NS = 16
_NW = _NC * _NS
_G = 4
_BC = 16
_GPC = _BC // _G
_PD = _NPAIRS * _EMBED
_ROWW = _FIELDS * _EMBED


def _pairs_for_batch(rows_ref, ostage_ref, gb, lb_vec, iota16):
    rbase = gb * _FIELDS

    def body(p, carry):
        i, j = carry
        a = rows_ref[rbase + i, j, :]
        b = rows_ref[rbase + j, i, :]
        pd_vec = iota16 + p * _EMBED
        plsc.store_scatter(ostage_ref, [pd_vec, lb_vec], a * b)
        last = j == (_FIELDS - 1)
        i2 = jnp.where(last, i + 1, i)
        j2 = jnp.where(last, i + 2, j + 1)
        return (i2, j2)

    lax.fori_loop(0, _NPAIRS, body, (jnp.int32(0), jnp.int32(1)))


def _sc_body(nb, nchunk, x_hbm, v_hbm, out_hbm,
             idx_v, rows_v, ostage, gsem, osem):
    wid = lax.axis_index("s") * _NC + lax.axis_index("c")
    base = wid * nb
    iota16 = lax.iota(jnp.int32, _EMBED)

    def out_copy(c):
        return pltpu.make_async_copy(
            ostage, out_hbm.at[:, pl.ds((base + c * _BC), _BC)], osem)

    def chunk_body(c, carry):
        for lg in range(_GPC):
            g = c * _GPC + lg
            pltpu.sync_copy(
                x_hbm.at[pl.ds((base + g * _G) * _FIELDS, _G * _FIELDS)],
                idx_v)
            gather = pltpu.make_async_copy(v_hbm.at[idx_v], rows_v, gsem)
            gather.start()
            if lg == 0:
                @pl.when(c > 0)
                def _():
                    out_copy(c - 1).wait()
            gather.wait()

            def inner(gb, cc):
                lb_vec = jnp.broadcast_to(lg * _G + gb, (_EMBED,))
                _pairs_for_batch(rows_v, ostage, gb, lb_vec, iota16)
                return cc

            lax.fori_loop(0, _G, inner, 0)
        out_copy(c).start()
        return carry

    lax.fori_loop(0, nchunk, chunk_body, 0)
    out_copy(nchunk - 1).wait()


_TS = 32
_NSTRIP = -(-100000 // _TS)
_TPW = 2 * -(-_NSTRIP // (2 * _NW))


def _t_body(vmax, vt_hbm, t2_hbm, stage0, stage1, trans0, trans1,
            isem0, isem1, osem0, osem1):
    stage = (stage0, stage1)
    trans = (trans0, trans1)
    isem = (isem0, isem1)
    osem = (osem0, osem1)
    wid = lax.axis_index("s") * _NC + lax.axis_index("c")
    iota416 = lax.iota(jnp.int32, _EMBED) * _ROWW

    def r0_of(t):
        return jnp.minimum((wid + t * _NW) * _TS, vmax - _TS)

    def in_copy(t, buf):
        return pltpu.make_async_copy(
            vt_hbm.at[:, :, pl.ds(r0_of(t), _TS)], stage[buf], isem[buf])

    def out_copy(t, buf):
        return pltpu.make_async_copy(
            trans[buf],
            t2_hbm.at[pl.ds(r0_of(t) * _ROWW, _TS * _ROWW)],
            osem[buf])

    in_copy(0, 0).start()

    def outer(tt, carry):
        for b in (0, 1):
            t = tt * 2 + b

            @pl.when(t + 1 < _TPW)
            def _():
                in_copy(t + 1, (b + 1) % 2).start()

            in_copy(t, b).wait()

            @pl.when(t >= 2)
            def _():
                out_copy(t - 2, b).wait()

            def per_field(f, cc):
                fbase = f * _EMBED
                for d in range(_EMBED):
                    for rc in range(_TS // _EMBED):
                        vals = stage[b][f, d, pl.ds(rc * _EMBED, _EMBED)]
                        idx = iota416 + (fbase + (rc * _EMBED * _ROWW + d))
                        plsc.store_scatter(trans[b], [idx], vals)
                return cc

            lax.fori_loop(0, _FIELDS, per_field, 0)
            out_copy(t, b).start()
        return carry

    lax.fori_loop(0, _TPW // 2, outer, 0)
    out_copy(_TPW - 2, 0).wait()
    out_copy(_TPW - 1, 1).wait()


def kernel(X, v):
    B, F = X.shape
    Vn, F2, D = v.shape
    assert F == _FIELDS and F2 == _FIELDS and D == _EMBED
    assert B % (_NW * _BC) == 0
    nb = B // _NW
    nchunk = nb // _BC

    x_flat = X.reshape(B * F).astype(jnp.int32)
    vt = jnp.transpose(v, (1, 2, 0))

    mesh = plsc.VectorSubcoreMesh(core_axis_name="c", subcore_axis_name="s")
    f32 = jnp.float32
    run_t = pl.kernel(
        functools.partial(_t_body, Vn),
        mesh=mesh,
        compiler_params=pltpu.CompilerParams(
            use_tc_tiling_on_sc=False, needs_layout_passes=False),
        out_type=jax.ShapeDtypeStruct((Vn * F * D,), f32),
        scratch_types=[
            pltpu.VMEM((_FIELDS, _EMBED, _TS), f32),
            pltpu.VMEM((_FIELDS, _EMBED, _TS), f32),
            pltpu.VMEM((_TS * _FIELDS * _EMBED,), f32),
            pltpu.VMEM((_TS * _FIELDS * _EMBED,), f32),
            pltpu.SemaphoreType.DMA,
            pltpu.SemaphoreType.DMA,
            pltpu.SemaphoreType.DMA,
            pltpu.SemaphoreType.DMA,
        ],
    )
    run = pl.kernel(
        functools.partial(_sc_body, nb, nchunk),
        mesh=mesh,
        compiler_params=pltpu.CompilerParams(
            use_tc_tiling_on_sc=False, needs_layout_passes=False),
        out_type=jax.ShapeDtypeStruct((_PD, B), f32),
        scratch_types=[
            pltpu.VMEM((_G * _FIELDS,), jnp.int32),
            pltpu.VMEM((_G * _FIELDS, _FIELDS, _EMBED), f32),
            pltpu.VMEM((_PD, _BC), f32),
            pltpu.SemaphoreType.DMA,
            pltpu.SemaphoreType.DMA,
        ],
    )
    table2 = run_t(vt).reshape(Vn, F, D)
    out2 = run(x_flat, table2)
    return out2.reshape(_NPAIRS, _EMBED, B).transpose(2, 0, 1)
